# Initial kernel scaffold; baseline (speedup 1.0000x reference)
import jax
import jax.numpy as jnp
from jax.experimental import pallas as pl


def kernel(gene_emb, edge_index, W1, b1, W2, b2):
    raise NotImplementedError("write your pallas kernel here")


# speedup vs baseline: 17.5676x; 17.5676x over previous
import functools

import jax
import jax.numpy as jnp
from jax import lax
from jax.experimental import pallas as pl
from jax.experimental.pallas import tpu as pltpu
from jax.experimental.pallas import tpu_sc as plsc

BB = 4
NN = 10000
EE = 160000
DD = 128

NC = 2
NS = 16
LANES = 16

NPAD = 10240
EPT = 10240
CK = 128
NCH = EPT // CK
ROWS_PT = NPAD // NS
NQ = ROWS_PT // CK

_mesh = plsc.VectorSubcoreMesh(core_axis_name="c", subcore_axis_name="s")


@functools.partial(
    pl.kernel,
    out_type=jax.ShapeDtypeStruct((NPAD,), jnp.float32),
    mesh=_mesh,
    scratch_types=[
        pltpu.VMEM((NCH, CK), jnp.int32),
        pltpu.VMEM((CK,), jnp.float32),
        pltpu.VMEM((ROWS_PT,), jnp.float32),
        pltpu.VMEM_SHARED((NPAD,), jnp.float32),
    ],
)
def _k1_deg(dst3, deg_out, dst2, ones, zbuf, deg_sh):
    c = lax.axis_index("c")
    s = lax.axis_index("s")
    active = c == 0
    row0 = s * ROWS_PT

    @pl.when(active)
    def _prep():
        def zb(i, _):
            zbuf[pl.ds(i * LANES, LANES)] = jnp.zeros((LANES,), jnp.float32)
            return 0
        lax.fori_loop(0, ROWS_PT // LANES, zb, 0)
        pltpu.sync_copy(zbuf, deg_sh.at[pl.ds(row0, ROWS_PT)])
        pltpu.sync_copy(dst3.at[s], dst2)
        def vb(i, _):
            ones[pl.ds(i * LANES, LANES)] = jnp.ones((LANES,), jnp.float32)
            return 0
        lax.fori_loop(0, CK // LANES, vb, 0)
        plsc.subcore_barrier()

        def body(j, _):
            pltpu.sync_copy(ones, deg_sh.at[dst2.at[j]], add=True)
            return 0
        lax.fori_loop(0, NCH, body, 0)
        plsc.subcore_barrier()

        pltpu.sync_copy(deg_sh.at[pl.ds(row0, ROWS_PT)], zbuf)
        pltpu.sync_copy(zbuf, deg_out.at[pl.ds(row0, ROWS_PT)])


_TCB = 1024
_NTB = NPAD // _TCB


def _k2_body(x_ref, w_ref, deg_ref, y_ref, dinv_ref):
    xb = x_ref[0]
    dinv = lax.rsqrt(deg_ref[0, 0] + 1.0)
    dinv_ref[0, 0] = dinv
    xw = jnp.dot(xb, w_ref[...], preferred_element_type=jnp.float32)
    y_ref[0] = xw * dinv[:, None]


def _k2_y(xpad, W1, deg2):
    return pl.pallas_call(
        _k2_body,
        out_shape=(
            jax.ShapeDtypeStruct((BB, NPAD, DD), jnp.float32),
            jax.ShapeDtypeStruct((_NTB, 1, _TCB), jnp.float32),
        ),
        grid=(BB, _NTB),
        in_specs=[
            pl.BlockSpec((1, _TCB, DD), lambda b, i: (b, i, 0)),
            pl.BlockSpec((DD, DD), lambda b, i: (0, 0)),
            pl.BlockSpec((1, 1, _TCB), lambda b, i: (i, 0, 0)),
        ],
        out_specs=(
            pl.BlockSpec((1, _TCB, DD), lambda b, i: (b, i, 0)),
            pl.BlockSpec((1, 1, _TCB), lambda b, i: (i, 0, 0)),
        ),
    )(xpad, W1, deg2)


@functools.partial(
    pl.kernel,
    out_type=(
        jax.ShapeDtypeStruct((BB, NPAD, DD), jnp.float32),
        jax.ShapeDtypeStruct((NPAD,), jnp.float32),
    ),
    mesh=_mesh,
    scratch_types=[
        pltpu.VMEM((NCH, CK), jnp.int32),
        pltpu.VMEM((NCH, CK), jnp.int32),
        pltpu.VMEM((CK,), jnp.float32),
        pltpu.VMEM((ROWS_PT,), jnp.float32),
        pltpu.VMEM((CK, DD), jnp.float32),
        pltpu.VMEM_SHARED((NPAD, DD), jnp.float32),
        pltpu.VMEM_SHARED((NPAD,), jnp.float32),
    ],
)
def _k3_scatter(y2, srcg, dst3, src3, dinv_h, s_out, t_out,
                src2, dst2, valbuf, zb1, gbuf,
                agg_sh, t_sh):
    c = lax.axis_index("c")
    s = lax.axis_index("s")
    row0 = s * ROWS_PT

    pltpu.sync_copy(dst3.at[s], dst2)

    @pl.when(c == 0)
    def _t_phase():
        def zb1f(i, _):
            zb1[pl.ds(i * LANES, LANES)] = jnp.zeros((LANES,), jnp.float32)
            return 0
        lax.fori_loop(0, ROWS_PT // LANES, zb1f, 0)
        pltpu.sync_copy(zb1, t_sh.at[pl.ds(row0, ROWS_PT)])
        pltpu.sync_copy(src3.at[s], src2)
        plsc.subcore_barrier()

        def body(j, _):
            pltpu.sync_copy(dinv_h.at[dst2.at[j]], valbuf)
            pltpu.sync_copy(valbuf, t_sh.at[src2.at[j]], add=True)
            return 0
        lax.fori_loop(0, NCH, body, 0)
        plsc.subcore_barrier()

        pltpu.sync_copy(t_sh.at[pl.ds(row0, ROWS_PT)], zb1)
        pltpu.sync_copy(zb1, t_out.at[pl.ds(row0, ROWS_PT)])

    for b_i in range(BB // NC):
        b = c * (BB // NC) + b_i
        pltpu.sync_copy(srcg.at[b, s], src2)
        def zb(i, _):
            r = i // (DD // LANES)
            u = i % (DD // LANES)
            gbuf[r, pl.ds(u * LANES, LANES)] = jnp.zeros((LANES,), jnp.float32)
            return 0
        lax.fori_loop(0, CK * (DD // LANES), zb, 0)
        for q in range(NQ):
            pltpu.sync_copy(gbuf, agg_sh.at[pl.ds(row0 + q * CK, CK)])

        plsc.subcore_barrier()

        def body(j, _):
            pltpu.sync_copy(y2.at[src2.at[j]], gbuf)
            pltpu.sync_copy(gbuf, agg_sh.at[dst2.at[j]], add=True)
            return 0
        lax.fori_loop(0, NCH, body, 0)

        plsc.subcore_barrier()

        for q in range(NQ):
            r = row0 + q * CK
            pltpu.sync_copy(agg_sh.at[pl.ds(r, CK)], gbuf)
            pltpu.sync_copy(gbuf, s_out.at[b].at[pl.ds(r, CK)])

        plsc.subcore_barrier()


def _k4_body(s_ref, y_ref, dinv_ref, t_ref, b1_ref, w2_ref, b2_ref,
             out_ref, acc_ref):
    i = pl.program_id(0)

    @pl.when(i == 0)
    def _init():
        acc_ref[...] = jnp.zeros_like(acc_ref)

    dinv = dinv_ref[0, 0]
    cb = dinv * (dinv + t_ref[0, 0])
    rows = i * _TCB + lax.broadcasted_iota(jnp.int32, (_TCB,), 0)
    cb = jnp.where(rows < NN, cb, 0.0)
    h = (s_ref[...] + y_ref[...]) * dinv[None, :, None] + b1_ref[0][None, None, :]
    h = jnp.maximum(h, 0.0)
    acc_ref[...] += jnp.sum(h * cb[None, :, None], axis=1)

    @pl.when(i == _NTB - 1)
    def _fin():
        out_ref[...] = (
            jnp.dot(acc_ref[...] * (1.0 / NN), w2_ref[...],
                    preferred_element_type=jnp.float32)
            + b2_ref[...]
        )


def _k4_reduce(S, y3, dinv2, t2, b1r, W2, b2r):
    return pl.pallas_call(
        _k4_body,
        out_shape=jax.ShapeDtypeStruct((BB, DD), jnp.float32),
        grid=(_NTB,),
        in_specs=[
            pl.BlockSpec((BB, _TCB, DD), lambda i: (0, i, 0)),
            pl.BlockSpec((BB, _TCB, DD), lambda i: (0, i, 0)),
            pl.BlockSpec((1, 1, _TCB), lambda i: (i, 0, 0)),
            pl.BlockSpec((1, 1, _TCB), lambda i: (i, 0, 0)),
            pl.BlockSpec((1, DD), lambda i: (0, 0)),
            pl.BlockSpec((DD, DD), lambda i: (0, 0)),
            pl.BlockSpec((1, DD), lambda i: (0, 0)),
        ],
        out_specs=pl.BlockSpec((BB, DD), lambda i: (0, 0)),
        scratch_shapes=[pltpu.VMEM((BB, DD), jnp.float32)],
    )(S, y3, dinv2, t2, b1r, W2, b2r)


def kernel(gene_emb, edge_index, W1, b1, W2, b2):
    src = edge_index[0].astype(jnp.int32)
    dst = edge_index[1].astype(jnp.int32)
    dummy = NPAD - 1
    ept_real = EE // NS

    src_t = jnp.pad(src.reshape(NS, ept_real), ((0, 0), (0, EPT - ept_real)),
                    constant_values=dummy)
    dst_t = jnp.pad(dst.reshape(NS, ept_real), ((0, 0), (0, EPT - ept_real)),
                    constant_values=dummy)
    src3 = src_t.reshape(NS, NCH, CK)
    dst3 = dst_t.reshape(NS, NCH, CK)
    offs = (jnp.arange(BB, dtype=jnp.int32) * NPAD)[:, None, None]
    srcg = (src_t[None] + offs).reshape(BB, NS, NCH, CK)

    deg = _k1_deg(dst3)
    deg2 = deg.reshape(_NTB, 1, _TCB)

    xpad = jnp.pad(gene_emb, ((0, 0), (0, NPAD - NN), (0, 0)))
    y3, dinv2 = _k2_y(xpad, W1, deg2)
    dinv = dinv2.reshape(NPAD)

    S, t = _k3_scatter(y3.reshape(BB * NPAD, DD), srcg, dst3, src3, dinv)
    t2 = t.reshape(_NTB, 1, _TCB)

    out = _k4_reduce(S, y3, dinv2, t2, b1.reshape(1, DD), W2,
                     b2.reshape(1, DD))
    return out


# speedup vs baseline: 23.5937x; 1.3430x over previous
import functools

import jax
import jax.numpy as jnp
from jax import lax
from jax.experimental import pallas as pl
from jax.experimental.pallas import tpu as pltpu
from jax.experimental.pallas import tpu_sc as plsc

BB = 4
NN = 10000
EE = 160000
DD = 128

NC = 2
NS = 16
LANES = 16

NPAD = 10240
CK = 128
NCH = 80
HB = NCH // 2
EPT = NCH * CK
ROWS_PT = NPAD // NS
WCK = 128
NQ = ROWS_PT // WCK
TH = NCH // NC

_mesh = plsc.VectorSubcoreMesh(core_axis_name="c", subcore_axis_name="s")


@functools.partial(
    pl.kernel,
    out_type=jax.ShapeDtypeStruct((NPAD,), jnp.float32),
    mesh=_mesh,
    scratch_types=[
        pltpu.VMEM((NCH, CK), jnp.int32),
        pltpu.VMEM((CK,), jnp.float32),
        pltpu.VMEM((ROWS_PT,), jnp.float32),
        pltpu.VMEM_SHARED((NPAD,), jnp.float32),
    ],
)
def _k1_deg(dst3, deg_out, dst2, ones, zbuf, deg_sh):
    c = lax.axis_index("c")
    s = lax.axis_index("s")
    active = c == 0
    row0 = s * ROWS_PT

    @pl.when(active)
    def _prep():
        def zb(i, _):
            zbuf[pl.ds(i * LANES, LANES)] = jnp.zeros((LANES,), jnp.float32)
            return 0
        lax.fori_loop(0, ROWS_PT // LANES, zb, 0)
        pltpu.sync_copy(zbuf, deg_sh.at[pl.ds(row0, ROWS_PT)])
        pltpu.sync_copy(dst3.at[s], dst2)
        def vb(i, _):
            ones[pl.ds(i * LANES, LANES)] = jnp.ones((LANES,), jnp.float32)
            return 0
        lax.fori_loop(0, CK // LANES, vb, 0)
        plsc.subcore_barrier()

        def body(j, _):
            pltpu.sync_copy(ones, deg_sh.at[dst2.at[j]], add=True)
            return 0
        lax.fori_loop(0, NCH, body, 0)
        plsc.subcore_barrier()

        pltpu.sync_copy(deg_sh.at[pl.ds(row0, ROWS_PT)], zbuf)
        pltpu.sync_copy(zbuf, deg_out.at[pl.ds(row0, ROWS_PT)])


_TCB = 1024
_NTB = NPAD // _TCB


def _k2_body(x_ref, w_ref, deg_ref, y_ref, dinv_ref):
    xb = x_ref[0]
    dinv = lax.rsqrt(deg_ref[0, 0] + 1.0)
    dinv_ref[0, 0] = dinv
    xw = jnp.dot(xb, w_ref[...], preferred_element_type=jnp.float32)
    y_ref[0] = xw * dinv[:, None]


def _k2_y(xpad, W1, deg2):
    return pl.pallas_call(
        _k2_body,
        out_shape=(
            jax.ShapeDtypeStruct((BB, NPAD, DD), jnp.float32),
            jax.ShapeDtypeStruct((_NTB, 1, _TCB), jnp.float32),
        ),
        grid=(BB, _NTB),
        in_specs=[
            pl.BlockSpec((1, _TCB, DD), lambda b, i: (b, i, 0)),
            pl.BlockSpec((DD, DD), lambda b, i: (0, 0)),
            pl.BlockSpec((1, 1, _TCB), lambda b, i: (i, 0, 0)),
        ],
        out_specs=(
            pl.BlockSpec((1, _TCB, DD), lambda b, i: (b, i, 0)),
            pl.BlockSpec((1, 1, _TCB), lambda b, i: (i, 0, 0)),
        ),
    )(xpad, W1, deg2)


@functools.partial(
    pl.kernel,
    out_type=(
        jax.ShapeDtypeStruct((BB, NPAD, DD), jnp.float32),
        jax.ShapeDtypeStruct((NC, NPAD), jnp.float32),
    ),
    mesh=_mesh,
    scratch_types=[
        pltpu.VMEM((HB, CK), jnp.int32),
        pltpu.VMEM((HB, CK), jnp.int32),
        pltpu.VMEM((CK,), jnp.float32),
        pltpu.VMEM((CK,), jnp.float32),
        pltpu.VMEM((ROWS_PT,), jnp.float32),
        pltpu.VMEM((CK, DD), jnp.float32),
        pltpu.VMEM((CK, DD), jnp.float32),
        pltpu.VMEM_SHARED((NPAD, DD), jnp.float32),
        pltpu.VMEM_SHARED((NPAD,), jnp.float32),
        pltpu.SemaphoreType.DMA,
        pltpu.SemaphoreType.DMA,
        pltpu.SemaphoreType.DMA,
        pltpu.SemaphoreType.DMA,
    ],
)
def _k3_scatter(y2, srcg, dst3, src3, dinv_h, s_out, t_out,
                src2, dst2, val0, val1, zb1, gbuf0, gbuf1,
                agg_sh, t_sh, gs0, gs1, ss0, ss1):
    c = lax.axis_index("c")
    s = lax.axis_index("s")
    row0 = s * ROWS_PT

    pltpu.sync_copy(dst3.at[s, pl.ds(c * TH, TH)], dst2)
    pltpu.sync_copy(src3.at[s, pl.ds(c * TH, TH)], src2)
    def zb1f(i, _):
        zb1[pl.ds(i * LANES, LANES)] = jnp.zeros((LANES,), jnp.float32)
        return 0
    lax.fori_loop(0, ROWS_PT // LANES, zb1f, 0)
    pltpu.sync_copy(zb1, t_sh.at[pl.ds(row0, ROWS_PT)])
    plsc.subcore_barrier()

    pltpu.async_copy(dinv_h.at[dst2.at[0]], val0, gs0)

    def t_body(m, _):
        j = 2 * m
        pltpu.make_async_copy(dinv_h.at[dst2.at[j]], val0, gs0).wait()
        @pl.when(m > 0)
        def _ws1():
            pltpu.make_async_copy(val1, t_sh.at[src2.at[j - 1]], ss1).wait()
        pltpu.async_copy(dinv_h.at[dst2.at[j + 1]], val1, gs1)
        pltpu.async_copy(val0, t_sh.at[src2.at[j]], ss0, add=True)
        pltpu.make_async_copy(dinv_h.at[dst2.at[j + 1]], val1, gs1).wait()
        pltpu.make_async_copy(val0, t_sh.at[src2.at[j]], ss0).wait()
        @pl.when(j + 2 < TH)
        def _g0():
            pltpu.async_copy(dinv_h.at[dst2.at[j + 2]], val0, gs0)
        pltpu.async_copy(val1, t_sh.at[src2.at[j + 1]], ss1, add=True)
        return 0
    lax.fori_loop(0, TH // 2, t_body, 0)
    pltpu.make_async_copy(val1, t_sh.at[src2.at[TH - 1]], ss1).wait()

    plsc.subcore_barrier()
    pltpu.sync_copy(t_sh.at[pl.ds(row0, ROWS_PT)], zb1)
    pltpu.sync_copy(zb1, t_out.at[c].at[pl.ds(row0, ROWS_PT)])

    for b_i in range(BB // NC):
        b = c * (BB // NC) + b_i
        def zb(i, _):
            r = i // (DD // LANES)
            u = i % (DD // LANES)
            gbuf0[r, pl.ds(u * LANES, LANES)] = jnp.zeros((LANES,), jnp.float32)
            return 0
        lax.fori_loop(0, CK * (DD // LANES), zb, 0)
        for q in range(NQ):
            pltpu.sync_copy(gbuf0.at[pl.ds(0, WCK)],
                            agg_sh.at[pl.ds(row0 + q * WCK, WCK)])

        plsc.subcore_barrier()

        for h in range(2):
            pltpu.sync_copy(srcg.at[b, s, pl.ds(h * HB, HB)], src2)
            pltpu.sync_copy(dst3.at[s, pl.ds(h * HB, HB)], dst2)
            pltpu.async_copy(y2.at[src2.at[0]], gbuf0, gs0)

            def body(m, _):
                j = 2 * m
                pltpu.make_async_copy(y2.at[src2.at[j]], gbuf0, gs0).wait()
                @pl.when(m > 0)
                def _ws1():
                    pltpu.make_async_copy(gbuf1, agg_sh.at[dst2.at[j - 1]], ss1).wait()
                pltpu.async_copy(y2.at[src2.at[j + 1]], gbuf1, gs1)
                pltpu.async_copy(gbuf0, agg_sh.at[dst2.at[j]], ss0, add=True)
                pltpu.make_async_copy(y2.at[src2.at[j + 1]], gbuf1, gs1).wait()
                pltpu.make_async_copy(gbuf0, agg_sh.at[dst2.at[j]], ss0).wait()
                @pl.when(j + 2 < HB)
                def _g0():
                    pltpu.async_copy(y2.at[src2.at[j + 2]], gbuf0, gs0)
                pltpu.async_copy(gbuf1, agg_sh.at[dst2.at[j + 1]], ss1, add=True)
                return 0
            lax.fori_loop(0, HB // 2, body, 0)
            pltpu.make_async_copy(gbuf1, agg_sh.at[dst2.at[HB - 1]], ss1).wait()

        plsc.subcore_barrier()

        for q in range(NQ):
            r = row0 + q * WCK
            pltpu.sync_copy(agg_sh.at[pl.ds(r, WCK)], gbuf0.at[pl.ds(0, WCK)])
            pltpu.sync_copy(gbuf0.at[pl.ds(0, WCK)], s_out.at[b].at[pl.ds(r, WCK)])

        plsc.subcore_barrier()


def _k4_body(s_ref, y_ref, dinv_ref, t0_ref, t1_ref, b1_ref, w2_ref, b2_ref,
             out_ref, acc_ref):
    i = pl.program_id(0)

    @pl.when(i == 0)
    def _init():
        acc_ref[...] = jnp.zeros_like(acc_ref)

    dinv = dinv_ref[0, 0]
    cb = dinv * (dinv + t0_ref[0, 0] + t1_ref[0, 0])
    rows = i * _TCB + lax.broadcasted_iota(jnp.int32, (_TCB,), 0)
    cb = jnp.where(rows < NN, cb, 0.0)
    h = (s_ref[...] + y_ref[...]) * dinv[None, :, None] + b1_ref[0][None, None, :]
    h = jnp.maximum(h, 0.0)
    acc_ref[...] += jnp.sum(h * cb[None, :, None], axis=1)

    @pl.when(i == _NTB - 1)
    def _fin():
        out_ref[...] = (
            jnp.dot(acc_ref[...] * (1.0 / NN), w2_ref[...],
                    preferred_element_type=jnp.float32)
            + b2_ref[...]
        )


def _k4_reduce(S, y3, dinv2, t0, t1, b1r, W2, b2r):
    return pl.pallas_call(
        _k4_body,
        out_shape=jax.ShapeDtypeStruct((BB, DD), jnp.float32),
        grid=(_NTB,),
        in_specs=[
            pl.BlockSpec((BB, _TCB, DD), lambda i: (0, i, 0)),
            pl.BlockSpec((BB, _TCB, DD), lambda i: (0, i, 0)),
            pl.BlockSpec((1, 1, _TCB), lambda i: (i, 0, 0)),
            pl.BlockSpec((1, 1, _TCB), lambda i: (i, 0, 0)),
            pl.BlockSpec((1, 1, _TCB), lambda i: (i, 0, 0)),
            pl.BlockSpec((1, DD), lambda i: (0, 0)),
            pl.BlockSpec((DD, DD), lambda i: (0, 0)),
            pl.BlockSpec((1, DD), lambda i: (0, 0)),
        ],
        out_specs=pl.BlockSpec((BB, DD), lambda i: (0, 0)),
        scratch_shapes=[pltpu.VMEM((BB, DD), jnp.float32)],
    )(S, y3, dinv2, t0, t1, b1r, W2, b2r)


def kernel(gene_emb, edge_index, W1, b1, W2, b2):
    src = edge_index[0].astype(jnp.int32)
    dst = edge_index[1].astype(jnp.int32)
    dummy = NPAD - 1
    ept_real = EE // NS

    src_t = jnp.pad(src.reshape(NS, ept_real), ((0, 0), (0, EPT - ept_real)),
                    constant_values=dummy)
    dst_t = jnp.pad(dst.reshape(NS, ept_real), ((0, 0), (0, EPT - ept_real)),
                    constant_values=dummy)
    src3 = src_t.reshape(NS, NCH, CK)
    dst3 = dst_t.reshape(NS, NCH, CK)
    offs = (jnp.arange(BB, dtype=jnp.int32) * NPAD)[:, None, None]
    srcg = (src_t[None] + offs).reshape(BB, NS, NCH, CK)

    deg = _k1_deg(dst3)
    deg2 = deg.reshape(_NTB, 1, _TCB)

    xpad = jnp.pad(gene_emb, ((0, 0), (0, NPAD - NN), (0, 0)))
    y3, dinv2 = _k2_y(xpad, W1, deg2)
    dinv = dinv2.reshape(NPAD)

    S, t = _k3_scatter(y3.reshape(BB * NPAD, DD), srcg, dst3, src3, dinv)
    t0 = t[0].reshape(_NTB, 1, _TCB)
    t1 = t[1].reshape(_NTB, 1, _TCB)

    out = _k4_reduce(S, y3, dinv2, t0, t1, b1.reshape(1, DD), W2,
                     b2.reshape(1, DD))
    return out


# speedup vs baseline: 23.7449x; 1.0064x over previous
import functools

import jax
import jax.numpy as jnp
from jax import lax
from jax.experimental import pallas as pl
from jax.experimental.pallas import tpu as pltpu
from jax.experimental.pallas import tpu_sc as plsc

BB = 4
NN = 10000
EE = 160000
DD = 128

NC = 2
NS = 16
LANES = 16

NPAD = 10240
CK = 128
NCH = 80
HB = NCH // 2
EPT = NCH * CK
ROWS_PT = NPAD // NS
WCK = 128
NQ = ROWS_PT // WCK
TH = NCH // NC

_mesh = plsc.VectorSubcoreMesh(core_axis_name="c", subcore_axis_name="s")


@functools.partial(
    pl.kernel,
    out_type=jax.ShapeDtypeStruct((NPAD,), jnp.float32),
    mesh=_mesh,
    scratch_types=[
        pltpu.VMEM((NCH, CK), jnp.int32),
        pltpu.VMEM((CK,), jnp.float32),
        pltpu.VMEM((ROWS_PT,), jnp.float32),
        pltpu.VMEM_SHARED((NPAD,), jnp.float32),
    ],
)
def _k1_deg(dst3, deg_out, dst2, ones, zbuf, deg_sh):
    c = lax.axis_index("c")
    s = lax.axis_index("s")
    active = c == 0
    row0 = s * ROWS_PT

    @pl.when(active)
    def _prep():
        def zb(i, _):
            zbuf[pl.ds(i * LANES, LANES)] = jnp.zeros((LANES,), jnp.float32)
            return 0
        lax.fori_loop(0, ROWS_PT // LANES, zb, 0)
        pltpu.sync_copy(zbuf, deg_sh.at[pl.ds(row0, ROWS_PT)])
        pltpu.sync_copy(dst3.at[s], dst2)
        def vb(i, _):
            ones[pl.ds(i * LANES, LANES)] = jnp.ones((LANES,), jnp.float32)
            return 0
        lax.fori_loop(0, CK // LANES, vb, 0)
        plsc.subcore_barrier()

        def body(j, _):
            pltpu.sync_copy(ones, deg_sh.at[dst2.at[j]], add=True)
            return 0
        lax.fori_loop(0, NCH, body, 0)
        plsc.subcore_barrier()

        pltpu.sync_copy(deg_sh.at[pl.ds(row0, ROWS_PT)], zbuf)
        pltpu.sync_copy(zbuf, deg_out.at[pl.ds(row0, ROWS_PT)])


_TCB = 1024
_NTB = NPAD // _TCB


def _k2_body(x_ref, w_ref, deg_ref, y_ref, dinv_ref):
    xb = x_ref[0]
    dinv = lax.rsqrt(deg_ref[0, 0] + 1.0)
    dinv_ref[0, 0] = dinv
    xw = jnp.dot(xb, w_ref[...], preferred_element_type=jnp.float32)
    y_ref[0] = xw * dinv[:, None]


def _k2_y(xpad, W1, deg2):
    return pl.pallas_call(
        _k2_body,
        out_shape=(
            jax.ShapeDtypeStruct((BB, NPAD, DD), jnp.float32),
            jax.ShapeDtypeStruct((_NTB, 1, _TCB), jnp.float32),
        ),
        grid=(BB, _NTB),
        in_specs=[
            pl.BlockSpec((1, _TCB, DD), lambda b, i: (b, i, 0)),
            pl.BlockSpec((DD, DD), lambda b, i: (0, 0)),
            pl.BlockSpec((1, 1, _TCB), lambda b, i: (i, 0, 0)),
        ],
        out_specs=(
            pl.BlockSpec((1, _TCB, DD), lambda b, i: (b, i, 0)),
            pl.BlockSpec((1, 1, _TCB), lambda b, i: (i, 0, 0)),
        ),
    )(xpad, W1, deg2)


@functools.partial(
    pl.kernel,
    out_type=(
        jax.ShapeDtypeStruct((BB, NPAD, DD), jnp.float32),
        jax.ShapeDtypeStruct((NC, NPAD), jnp.float32),
    ),
    mesh=_mesh,
    scratch_types=[
        pltpu.VMEM((HB, CK), jnp.int32),
        pltpu.VMEM((HB, CK), jnp.int32),
        pltpu.VMEM((CK,), jnp.float32),
        pltpu.VMEM((CK,), jnp.float32),
        pltpu.VMEM((ROWS_PT,), jnp.float32),
        pltpu.VMEM((CK, DD), jnp.float32),
        pltpu.VMEM((CK, DD), jnp.float32),
        pltpu.VMEM_SHARED((NPAD, DD), jnp.float32),
        pltpu.VMEM_SHARED((NPAD,), jnp.float32),
        pltpu.SemaphoreType.DMA,
        pltpu.SemaphoreType.DMA,
        pltpu.SemaphoreType.DMA,
        pltpu.SemaphoreType.DMA,
    ],
)
def _k3_scatter(y2, srcg, dst3, src3, dinv_h, s_out, t_out,
                src2, dst2, val0, val1, zb1, gbuf0, gbuf1,
                agg_sh, t_sh, gs0, gs1, ss0, ss1):
    c = lax.axis_index("c")
    s = lax.axis_index("s")
    row0 = s * ROWS_PT

    pltpu.sync_copy(dst3.at[s, pl.ds(c * TH, TH)], dst2)
    pltpu.sync_copy(src3.at[s, pl.ds(c * TH, TH)], src2)
    def zb1f(i, _):
        zb1[pl.ds(i * LANES, LANES)] = jnp.zeros((LANES,), jnp.float32)
        return 0
    lax.fori_loop(0, ROWS_PT // LANES, zb1f, 0)
    pltpu.sync_copy(zb1, t_sh.at[pl.ds(row0, ROWS_PT)])
    plsc.subcore_barrier()

    pltpu.async_copy(dinv_h.at[dst2.at[0]], val0, gs0)

    def t_body(m, _):
        j = 2 * m
        pltpu.make_async_copy(dinv_h.at[dst2.at[j]], val0, gs0).wait()
        @pl.when(m > 0)
        def _ws1():
            pltpu.make_async_copy(val1, t_sh.at[src2.at[j - 1]], ss1).wait()
        pltpu.async_copy(dinv_h.at[dst2.at[j + 1]], val1, gs1)
        pltpu.async_copy(val0, t_sh.at[src2.at[j]], ss0, add=True)
        pltpu.make_async_copy(dinv_h.at[dst2.at[j + 1]], val1, gs1).wait()
        pltpu.make_async_copy(val0, t_sh.at[src2.at[j]], ss0).wait()
        @pl.when(j + 2 < TH)
        def _g0():
            pltpu.async_copy(dinv_h.at[dst2.at[j + 2]], val0, gs0)
        pltpu.async_copy(val1, t_sh.at[src2.at[j + 1]], ss1, add=True)
        return 0
    lax.fori_loop(0, TH // 2, t_body, 0)
    pltpu.make_async_copy(val1, t_sh.at[src2.at[TH - 1]], ss1).wait()

    plsc.subcore_barrier()
    pltpu.sync_copy(t_sh.at[pl.ds(row0, ROWS_PT)],
                    t_out.at[c].at[pl.ds(row0, ROWS_PT)])

    for b_i in range(BB // NC):
        b = c * (BB // NC) + b_i
        def zb(i, _):
            r = i // (DD // LANES)
            u = i % (DD // LANES)
            gbuf0[r, pl.ds(u * LANES, LANES)] = jnp.zeros((LANES,), jnp.float32)
            return 0
        lax.fori_loop(0, CK * (DD // LANES), zb, 0)
        for q in range(NQ):
            pltpu.async_copy(gbuf0.at[pl.ds(0, WCK)],
                             agg_sh.at[pl.ds(row0 + q * WCK, WCK)], gs0)
        for q in range(NQ):
            pltpu.make_async_copy(gbuf0.at[pl.ds(0, WCK)],
                                  agg_sh.at[pl.ds(row0 + q * WCK, WCK)],
                                  gs0).wait()

        plsc.subcore_barrier()

        for h in range(2):
            pltpu.sync_copy(srcg.at[b, s, pl.ds(h * HB, HB)], src2)
            pltpu.sync_copy(dst3.at[s, pl.ds(h * HB, HB)], dst2)
            pltpu.async_copy(y2.at[src2.at[0]], gbuf0, gs0)

            def body(m, _):
                j = 2 * m
                pltpu.make_async_copy(y2.at[src2.at[j]], gbuf0, gs0).wait()
                @pl.when(m > 0)
                def _ws1():
                    pltpu.make_async_copy(gbuf1, agg_sh.at[dst2.at[j - 1]], ss1).wait()
                pltpu.async_copy(y2.at[src2.at[j + 1]], gbuf1, gs1)
                pltpu.async_copy(gbuf0, agg_sh.at[dst2.at[j]], ss0, add=True)
                pltpu.make_async_copy(y2.at[src2.at[j + 1]], gbuf1, gs1).wait()
                pltpu.make_async_copy(gbuf0, agg_sh.at[dst2.at[j]], ss0).wait()
                @pl.when(j + 2 < HB)
                def _g0():
                    pltpu.async_copy(y2.at[src2.at[j + 2]], gbuf0, gs0)
                pltpu.async_copy(gbuf1, agg_sh.at[dst2.at[j + 1]], ss1, add=True)
                return 0
            lax.fori_loop(0, HB // 2, body, 0)
            pltpu.make_async_copy(gbuf1, agg_sh.at[dst2.at[HB - 1]], ss1).wait()

        plsc.subcore_barrier()

        for q in range(NQ):
            r = row0 + q * WCK
            pltpu.async_copy(agg_sh.at[pl.ds(r, WCK)],
                             s_out.at[b].at[pl.ds(r, WCK)], gs0)
        for q in range(NQ):
            r = row0 + q * WCK
            pltpu.make_async_copy(agg_sh.at[pl.ds(r, WCK)],
                                  s_out.at[b].at[pl.ds(r, WCK)], gs0).wait()

        plsc.subcore_barrier()


def _k4_body(s_ref, y_ref, dinv_ref, t0_ref, t1_ref, b1_ref, w2_ref, b2_ref,
             out_ref, acc_ref):
    i = pl.program_id(0)

    @pl.when(i == 0)
    def _init():
        acc_ref[...] = jnp.zeros_like(acc_ref)

    dinv = dinv_ref[0, 0]
    cb = dinv * (dinv + t0_ref[0, 0] + t1_ref[0, 0])
    rows = i * _TCB + lax.broadcasted_iota(jnp.int32, (_TCB,), 0)
    cb = jnp.where(rows < NN, cb, 0.0)
    h = (s_ref[...] + y_ref[...]) * dinv[None, :, None] + b1_ref[0][None, None, :]
    h = jnp.maximum(h, 0.0)
    acc_ref[...] += jnp.sum(h * cb[None, :, None], axis=1)

    @pl.when(i == _NTB - 1)
    def _fin():
        out_ref[...] = (
            jnp.dot(acc_ref[...] * (1.0 / NN), w2_ref[...],
                    preferred_element_type=jnp.float32)
            + b2_ref[...]
        )


def _k4_reduce(S, y3, dinv2, t0, t1, b1r, W2, b2r):
    return pl.pallas_call(
        _k4_body,
        out_shape=jax.ShapeDtypeStruct((BB, DD), jnp.float32),
        grid=(_NTB,),
        in_specs=[
            pl.BlockSpec((BB, _TCB, DD), lambda i: (0, i, 0)),
            pl.BlockSpec((BB, _TCB, DD), lambda i: (0, i, 0)),
            pl.BlockSpec((1, 1, _TCB), lambda i: (i, 0, 0)),
            pl.BlockSpec((1, 1, _TCB), lambda i: (i, 0, 0)),
            pl.BlockSpec((1, 1, _TCB), lambda i: (i, 0, 0)),
            pl.BlockSpec((1, DD), lambda i: (0, 0)),
            pl.BlockSpec((DD, DD), lambda i: (0, 0)),
            pl.BlockSpec((1, DD), lambda i: (0, 0)),
        ],
        out_specs=pl.BlockSpec((BB, DD), lambda i: (0, 0)),
        scratch_shapes=[pltpu.VMEM((BB, DD), jnp.float32)],
    )(S, y3, dinv2, t0, t1, b1r, W2, b2r)


def kernel(gene_emb, edge_index, W1, b1, W2, b2):
    src = edge_index[0].astype(jnp.int32)
    dst = edge_index[1].astype(jnp.int32)
    dummy = NPAD - 1
    ept_real = EE // NS

    src_t = jnp.pad(src.reshape(NS, ept_real), ((0, 0), (0, EPT - ept_real)),
                    constant_values=dummy)
    dst_t = jnp.pad(dst.reshape(NS, ept_real), ((0, 0), (0, EPT - ept_real)),
                    constant_values=dummy)
    src3 = src_t.reshape(NS, NCH, CK)
    dst3 = dst_t.reshape(NS, NCH, CK)
    offs = (jnp.arange(BB, dtype=jnp.int32) * NPAD)[:, None, None]
    srcg = (src_t[None] + offs).reshape(BB, NS, NCH, CK)

    deg = _k1_deg(dst3)
    deg2 = deg.reshape(_NTB, 1, _TCB)

    xpad = jnp.pad(gene_emb, ((0, 0), (0, NPAD - NN), (0, 0)))
    y3, dinv2 = _k2_y(xpad, W1, deg2)
    dinv = dinv2.reshape(NPAD)

    S, t = _k3_scatter(y3.reshape(BB * NPAD, DD), srcg, dst3, src3, dinv)
    t0 = t[0].reshape(_NTB, 1, _TCB)
    t1 = t[1].reshape(_NTB, 1, _TCB)

    out = _k4_reduce(S, y3, dinv2, t0, t1, b1.reshape(1, DD), W2,
                     b2.reshape(1, DD))
    return out


# speedup vs baseline: 26.9553x; 1.1352x over previous
import functools

import jax
import jax.numpy as jnp
from jax import lax
from jax.experimental import pallas as pl
from jax.experimental.pallas import tpu as pltpu
from jax.experimental.pallas import tpu_sc as plsc

BB = 4
NN = 10000
EE = 160000
DD = 128

NC = 2
NS = 16
LANES = 16

NPAD = 10240
CK = 128
NCH = 80
HB = NCH // 2
EPT = NCH * CK
ROWS_PT = NPAD // NS
WCK = 128
NQ = ROWS_PT // WCK
TH = NCH // NC

_mesh = plsc.VectorSubcoreMesh(core_axis_name="c", subcore_axis_name="s")


@functools.partial(
    pl.kernel,
    out_type=jax.ShapeDtypeStruct((NPAD,), jnp.float32),
    mesh=_mesh,
    scratch_types=[
        pltpu.VMEM((NCH, CK), jnp.int32),
        pltpu.VMEM((CK,), jnp.float32),
        pltpu.VMEM((ROWS_PT,), jnp.float32),
        pltpu.VMEM_SHARED((NPAD,), jnp.float32),
    ],
)
def _k1_deg(dst3, deg_out, dst2, ones, zbuf, deg_sh):
    c = lax.axis_index("c")
    s = lax.axis_index("s")
    active = c == 0
    row0 = s * ROWS_PT

    @pl.when(active)
    def _prep():
        def zb(i, _):
            zbuf[pl.ds(i * LANES, LANES)] = jnp.zeros((LANES,), jnp.float32)
            return 0
        lax.fori_loop(0, ROWS_PT // LANES, zb, 0)
        pltpu.sync_copy(zbuf, deg_sh.at[pl.ds(row0, ROWS_PT)])
        pltpu.sync_copy(dst3.at[s], dst2)
        def vb(i, _):
            ones[pl.ds(i * LANES, LANES)] = jnp.ones((LANES,), jnp.float32)
            return 0
        lax.fori_loop(0, CK // LANES, vb, 0)
        plsc.subcore_barrier()

        def body(j, _):
            pltpu.sync_copy(ones, deg_sh.at[dst2.at[j]], add=True)
            return 0
        lax.fori_loop(0, NCH, body, 0)
        plsc.subcore_barrier()

        pltpu.sync_copy(deg_sh.at[pl.ds(row0, ROWS_PT)], zbuf)
        pltpu.sync_copy(zbuf, deg_out.at[pl.ds(row0, ROWS_PT)])


_TCB = 1024
_NTB = NPAD // _TCB


def _k2_body(x_ref, w_ref, deg_ref, y_ref, dinv_ref):
    xb = x_ref[0]
    dinv = lax.rsqrt(deg_ref[0, 0] + 1.0)
    dinv_ref[0, 0] = dinv
    xw = jnp.dot(xb, w_ref[...], preferred_element_type=jnp.float32)
    y_ref[0] = xw * dinv[:, None]


def _k2_y(xpad, W1, deg2):
    return pl.pallas_call(
        _k2_body,
        out_shape=(
            jax.ShapeDtypeStruct((BB, NPAD, DD), jnp.float32),
            jax.ShapeDtypeStruct((_NTB, 1, _TCB), jnp.float32),
        ),
        grid=(BB, _NTB),
        in_specs=[
            pl.BlockSpec((1, _TCB, DD), lambda b, i: (b, i, 0)),
            pl.BlockSpec((DD, DD), lambda b, i: (0, 0)),
            pl.BlockSpec((1, 1, _TCB), lambda b, i: (i, 0, 0)),
        ],
        out_specs=(
            pl.BlockSpec((1, _TCB, DD), lambda b, i: (b, i, 0)),
            pl.BlockSpec((1, 1, _TCB), lambda b, i: (i, 0, 0)),
        ),
    )(xpad, W1, deg2)


@functools.partial(
    pl.kernel,
    out_type=(
        jax.ShapeDtypeStruct((BB, NPAD, DD), jnp.float32),
        jax.ShapeDtypeStruct((NC, NPAD), jnp.float32),
    ),
    mesh=_mesh,
    scratch_types=[
        pltpu.VMEM((HB, CK), jnp.int32),
        pltpu.VMEM((HB, CK), jnp.int32),
        pltpu.VMEM((CK,), jnp.float32),
        pltpu.VMEM((CK,), jnp.float32),
        pltpu.VMEM((1, CK), jnp.int32),
        pltpu.VMEM((1, CK), jnp.int32),
        pltpu.VMEM((ROWS_PT,), jnp.float32),
        pltpu.VMEM((CK, DD), jnp.float32),
        pltpu.VMEM((CK, DD), jnp.float32),
        pltpu.VMEM_SHARED((NPAD, DD), jnp.float32),
        pltpu.VMEM_SHARED((NPAD,), jnp.float32),
        pltpu.SemaphoreType.DMA,
        pltpu.SemaphoreType.DMA,
        pltpu.SemaphoreType.DMA,
        pltpu.SemaphoreType.DMA,
        pltpu.SemaphoreType.DMA,
        pltpu.SemaphoreType.DMA,
        pltpu.SemaphoreType.DMA,
        pltpu.SemaphoreType.DMA,
    ],
)
def _k3_scatter(y2, srcg, dst3, src3, dinv_h, s_out, t_out,
                src2, dst2, val0, val1, ti0, ti1, zb1, gbuf0, gbuf1,
                agg_sh, t_sh, gs0, gs1, ss0, ss1, tg0, tg1, ts0, ts1):
    c = lax.axis_index("c")
    s = lax.axis_index("s")
    row0 = s * ROWS_PT

    def zb1f(i, _):
        zb1[pl.ds(i * LANES, LANES)] = jnp.zeros((LANES,), jnp.float32)
        return 0
    lax.fori_loop(0, ROWS_PT // LANES, zb1f, 0)
    pltpu.sync_copy(zb1, t_sh.at[pl.ds(row0, ROWS_PT)])

    for b_i in range(BB // NC):
        b = c * (BB // NC) + b_i
        boff = b * NPAD
        fuse_t = b_i == 0
        def zb(i, _):
            r = i // (DD // LANES)
            u = i % (DD // LANES)
            gbuf0[r, pl.ds(u * LANES, LANES)] = jnp.zeros((LANES,), jnp.float32)
            return 0
        lax.fori_loop(0, CK * (DD // LANES), zb, 0)
        for q in range(NQ):
            pltpu.async_copy(gbuf0.at[pl.ds(0, WCK)],
                             agg_sh.at[pl.ds(row0 + q * WCK, WCK)], gs0)
        for q in range(NQ):
            pltpu.make_async_copy(gbuf0.at[pl.ds(0, WCK)],
                                  agg_sh.at[pl.ds(row0 + q * WCK, WCK)],
                                  gs0).wait()

        plsc.subcore_barrier()

        for h in range(2):
            pltpu.sync_copy(srcg.at[b, s, pl.ds(h * HB, HB)], src2)
            pltpu.sync_copy(dst3.at[s, pl.ds(h * HB, HB)], dst2)
            pltpu.async_copy(y2.at[src2.at[0]], gbuf0, gs0)
            t_on = fuse_t and h in (0, 1)

            def loc_idx(j, out_ref):
                def li(u, _):
                    out_ref[0, pl.ds(u * LANES, LANES)] = (
                        src2[j, pl.ds(u * LANES, LANES)] - boff)
                    return 0
                lax.fori_loop(0, CK // LANES, li, 0)

            if t_on:
                @pl.when(c == h)
                def _tprime():
                    pltpu.async_copy(dinv_h.at[dst2.at[0]], val0, tg0)

            def body(m, _):
                j = 2 * m
                pltpu.make_async_copy(y2.at[src2.at[j]], gbuf0, gs0).wait()
                @pl.when(m > 0)
                def _ws1():
                    pltpu.make_async_copy(gbuf1, agg_sh.at[dst2.at[j - 1]], ss1).wait()
                pltpu.async_copy(y2.at[src2.at[j + 1]], gbuf1, gs1)
                pltpu.async_copy(gbuf0, agg_sh.at[dst2.at[j]], ss0, add=True)
                if t_on:
                    @pl.when(c == h)
                    def _t_a():
                        pltpu.make_async_copy(dinv_h.at[dst2.at[j]], val0, tg0).wait()
                        @pl.when(m > 0)
                        def _wt1():
                            pltpu.make_async_copy(val1, t_sh.at[ti1.at[0]], ts1).wait()
                        pltpu.async_copy(dinv_h.at[dst2.at[j + 1]], val1, tg1)
                        loc_idx(j, ti0)
                        pltpu.async_copy(val0, t_sh.at[ti0.at[0]], ts0, add=True)
                pltpu.make_async_copy(y2.at[src2.at[j + 1]], gbuf1, gs1).wait()
                pltpu.make_async_copy(gbuf0, agg_sh.at[dst2.at[j]], ss0).wait()
                @pl.when(j + 2 < HB)
                def _g0():
                    pltpu.async_copy(y2.at[src2.at[j + 2]], gbuf0, gs0)
                pltpu.async_copy(gbuf1, agg_sh.at[dst2.at[j + 1]], ss1, add=True)
                if t_on:
                    @pl.when(c == h)
                    def _t_b():
                        pltpu.make_async_copy(dinv_h.at[dst2.at[j + 1]], val1, tg1).wait()
                        pltpu.make_async_copy(val0, t_sh.at[ti0.at[0]], ts0).wait()
                        @pl.when(j + 2 < HB)
                        def _tg0():
                            pltpu.async_copy(dinv_h.at[dst2.at[j + 2]], val0, tg0)
                        loc_idx(j + 1, ti1)
                        pltpu.async_copy(val1, t_sh.at[ti1.at[0]], ts1, add=True)
                return 0
            lax.fori_loop(0, HB // 2, body, 0)
            pltpu.make_async_copy(gbuf1, agg_sh.at[dst2.at[HB - 1]], ss1).wait()
            if t_on:
                @pl.when(c == h)
                def _tdrain():
                    pltpu.make_async_copy(val1, t_sh.at[ti1.at[0]], ts1).wait()

        plsc.subcore_barrier()

        if fuse_t:
            pltpu.sync_copy(t_sh.at[pl.ds(row0, ROWS_PT)],
                            t_out.at[c].at[pl.ds(row0, ROWS_PT)])

        for q in range(NQ):
            r = row0 + q * WCK
            pltpu.async_copy(agg_sh.at[pl.ds(r, WCK)],
                             s_out.at[b].at[pl.ds(r, WCK)], gs0)
        for q in range(NQ):
            r = row0 + q * WCK
            pltpu.make_async_copy(agg_sh.at[pl.ds(r, WCK)],
                                  s_out.at[b].at[pl.ds(r, WCK)], gs0).wait()

        plsc.subcore_barrier()


def _k4_body(s_ref, y_ref, dinv_ref, t0_ref, t1_ref, b1_ref, w2_ref, b2_ref,
             out_ref, acc_ref):
    i = pl.program_id(0)

    @pl.when(i == 0)
    def _init():
        acc_ref[...] = jnp.zeros_like(acc_ref)

    dinv = dinv_ref[0, 0]
    cb = dinv * (dinv + t0_ref[0, 0] + t1_ref[0, 0])
    rows = i * _TCB + lax.broadcasted_iota(jnp.int32, (_TCB,), 0)
    cb = jnp.where(rows < NN, cb, 0.0)
    h = (s_ref[...] + y_ref[...]) * dinv[None, :, None] + b1_ref[0][None, None, :]
    h = jnp.maximum(h, 0.0)
    acc_ref[...] += jnp.sum(h * cb[None, :, None], axis=1)

    @pl.when(i == _NTB - 1)
    def _fin():
        out_ref[...] = (
            jnp.dot(acc_ref[...] * (1.0 / NN), w2_ref[...],
                    preferred_element_type=jnp.float32)
            + b2_ref[...]
        )


def _k4_reduce(S, y3, dinv2, t0, t1, b1r, W2, b2r):
    return pl.pallas_call(
        _k4_body,
        out_shape=jax.ShapeDtypeStruct((BB, DD), jnp.float32),
        grid=(_NTB,),
        in_specs=[
            pl.BlockSpec((BB, _TCB, DD), lambda i: (0, i, 0)),
            pl.BlockSpec((BB, _TCB, DD), lambda i: (0, i, 0)),
            pl.BlockSpec((1, 1, _TCB), lambda i: (i, 0, 0)),
            pl.BlockSpec((1, 1, _TCB), lambda i: (i, 0, 0)),
            pl.BlockSpec((1, 1, _TCB), lambda i: (i, 0, 0)),
            pl.BlockSpec((1, DD), lambda i: (0, 0)),
            pl.BlockSpec((DD, DD), lambda i: (0, 0)),
            pl.BlockSpec((1, DD), lambda i: (0, 0)),
        ],
        out_specs=pl.BlockSpec((BB, DD), lambda i: (0, 0)),
        scratch_shapes=[pltpu.VMEM((BB, DD), jnp.float32)],
    )(S, y3, dinv2, t0, t1, b1r, W2, b2r)


def kernel(gene_emb, edge_index, W1, b1, W2, b2):
    src = edge_index[0].astype(jnp.int32)
    dst = edge_index[1].astype(jnp.int32)
    dummy = NPAD - 1
    ept_real = EE // NS

    src_t = jnp.pad(src.reshape(NS, ept_real), ((0, 0), (0, EPT - ept_real)),
                    constant_values=dummy)
    dst_t = jnp.pad(dst.reshape(NS, ept_real), ((0, 0), (0, EPT - ept_real)),
                    constant_values=dummy)
    src3 = src_t.reshape(NS, NCH, CK)
    dst3 = dst_t.reshape(NS, NCH, CK)
    offs = (jnp.arange(BB, dtype=jnp.int32) * NPAD)[:, None, None]
    srcg = (src_t[None] + offs).reshape(BB, NS, NCH, CK)

    deg = _k1_deg(dst3)
    deg2 = deg.reshape(_NTB, 1, _TCB)

    xpad = jnp.pad(gene_emb, ((0, 0), (0, NPAD - NN), (0, 0)))
    y3, dinv2 = _k2_y(xpad, W1, deg2)
    dinv = dinv2.reshape(NPAD)

    S, t = _k3_scatter(y3.reshape(BB * NPAD, DD), srcg, dst3, src3, dinv)
    t0 = t[0].reshape(_NTB, 1, _TCB)
    t1 = t[1].reshape(_NTB, 1, _TCB)

    out = _k4_reduce(S, y3, dinv2, t0, t1, b1.reshape(1, DD), W2,
                     b2.reshape(1, DD))
    return out
